# Initial kernel scaffold; baseline (speedup 1.0000x reference)
import functools

import jax
import jax.numpy as jnp
from jax import lax
from jax.experimental import pallas as pl
from jax.experimental.pallas import tpu as pltpu
from jax.experimental.pallas import tpu_sc as plsc

_N = 10000
_E = 320000
_D = 128
_ED = 16
_TDW = 32

_NC = 2
_NS = 16
_NW = _NC * _NS
_EPW = _E // _NW
_CHUNK = 80
_NCH = _EPW // _CHUNK
_RPT = _N // _NS

_mesh = plsc.VectorSubcoreMesh(core_axis_name="c", subcore_axis_name="s")


def _sc_scatter1(h_hbm, src_hbm, dst_hbm, ep_hbm, z128_hbm, z32_hbm,
                 s_out, td_out,
                 src_v, dst_v, rows_v, ep_v, sem, acc_s, acc_td):
    c = lax.axis_index("c")
    s = lax.axis_index("s")
    w = c * _NS + s
    r0 = s * _RPT
    pltpu.sync_copy(z128_hbm.at[pl.ds(r0, _RPT)], acc_s.at[pl.ds(r0, _RPT)])
    pltpu.sync_copy(z32_hbm.at[pl.ds(r0, _RPT)], acc_td.at[pl.ds(r0, _RPT)])
    pltpu.sync_copy(src_hbm.at[w], src_v)
    pltpu.sync_copy(dst_hbm.at[w], dst_v)
    plsc.subcore_barrier()

    def chunk(j, carry):
        pltpu.async_copy(h_hbm.at[src_v.at[j]], rows_v, sem).wait()
        pltpu.sync_copy(rows_v, acc_s.at[dst_v.at[j]], add=True)
        pltpu.async_copy(ep_hbm.at[w * _NCH + j], ep_v, sem).wait()
        pltpu.sync_copy(ep_v, acc_td.at[dst_v.at[j]], add=True)
        return carry

    lax.fori_loop(0, _NCH, chunk, 0)
    plsc.subcore_barrier()
    pltpu.sync_copy(acc_s.at[pl.ds(r0, _RPT)], s_out.at[w])
    pltpu.sync_copy(acc_td.at[pl.ds(r0, _RPT)], td_out.at[w])


def _sc_scatter2(h_hbm, src_hbm, dst_hbm, z128_hbm,
                 s_out,
                 src_v, dst_v, rows_v, sem, acc_s):
    c = lax.axis_index("c")
    s = lax.axis_index("s")
    w = c * _NS + s
    r0 = s * _RPT
    pltpu.sync_copy(z128_hbm.at[pl.ds(r0, _RPT)], acc_s.at[pl.ds(r0, _RPT)])
    pltpu.sync_copy(src_hbm.at[w], src_v)
    pltpu.sync_copy(dst_hbm.at[w], dst_v)
    plsc.subcore_barrier()

    def chunk(j, carry):
        pltpu.async_copy(h_hbm.at[src_v.at[j]], rows_v, sem).wait()
        pltpu.sync_copy(rows_v, acc_s.at[dst_v.at[j]], add=True)
        return carry

    lax.fori_loop(0, _NCH, chunk, 0)
    plsc.subcore_barrier()
    pltpu.sync_copy(acc_s.at[pl.ds(r0, _RPT)], s_out.at[w])


_scatter1 = functools.partial(
    pl.kernel,
    out_type=(jax.ShapeDtypeStruct((_NW, _RPT, _D), jnp.float32),
              jax.ShapeDtypeStruct((_NW, _RPT, _TDW), jnp.float32)),
    mesh=_mesh,
    scratch_types=(
        pltpu.VMEM((_NCH, _CHUNK), jnp.int32),
        pltpu.VMEM((_NCH, _CHUNK), jnp.int32),
        pltpu.VMEM((_CHUNK, _D), jnp.float32),
        pltpu.VMEM((_CHUNK, _TDW), jnp.float32),
        pltpu.SemaphoreType.DMA,
        pltpu.VMEM_SHARED((_N, _D), jnp.float32),
        pltpu.VMEM_SHARED((_N, _TDW), jnp.float32),
    ),
)(_sc_scatter1)

_scatter2 = functools.partial(
    pl.kernel,
    out_type=jax.ShapeDtypeStruct((_NW, _RPT, _D), jnp.float32),
    mesh=_mesh,
    scratch_types=(
        pltpu.VMEM((_NCH, _CHUNK), jnp.int32),
        pltpu.VMEM((_NCH, _CHUNK), jnp.int32),
        pltpu.VMEM((_CHUNK, _D), jnp.float32),
        pltpu.SemaphoreType.DMA,
        pltpu.VMEM_SHARED((_N, _D), jnp.float32),
    ),
)(_sc_scatter2)


def _tc_apply_body(sp_ref, tdp_ref, h_ref, wm_ref, bm_ref, wa_ref, ba_ref,
                   out_ref):
    s_sum = sp_ref[0] + sp_ref[1]
    td = tdp_ref[0] + tdp_ref[1]
    t = td[:, :_ED]
    deg = td[:, _ED:_ED + 1]
    agg = (jnp.dot(s_sum, wm_ref[:_D, :], preferred_element_type=jnp.float32)
           + jnp.dot(t, wm_ref[_D:, :], preferred_element_type=jnp.float32)
           + deg * bm_ref[...])
    h_neigh = agg / jnp.maximum(deg, 1.0)
    out = (jnp.dot(h_ref[...], wa_ref[:_D, :],
                   preferred_element_type=jnp.float32)
           + jnp.dot(h_neigh, wa_ref[_D:, :],
                     preferred_element_type=jnp.float32)
           + ba_ref[...])
    out_ref[...] = jnp.maximum(out, 0.0)


_TC_B = 1250


def _tc_apply(s_part, td_part, h, wm, bm, wa, ba):
    grid = (_N // _TC_B,)
    return pl.pallas_call(
        _tc_apply_body,
        grid=grid,
        in_specs=[
            pl.BlockSpec((_NC, _TC_B, _D), lambda i: (0, i, 0)),
            pl.BlockSpec((_NC, _TC_B, _TDW), lambda i: (0, i, 0)),
            pl.BlockSpec((_TC_B, _D), lambda i: (i, 0)),
            pl.BlockSpec((_D + _ED, _D), lambda i: (0, 0)),
            pl.BlockSpec((1, _D), lambda i: (0, 0)),
            pl.BlockSpec((2 * _D, _D), lambda i: (0, 0)),
            pl.BlockSpec((1, _D), lambda i: (0, 0)),
        ],
        out_specs=pl.BlockSpec((_TC_B, _D), lambda i: (i, 0)),
        out_shape=jax.ShapeDtypeStruct((_N, _D), jnp.float32),
    )(s_part, td_part, h, wm, bm, wa, ba)


def kernel(nfeats, edge_index, efeats, Wm1, bm1, Wa1, ba1, Wm2, bm2, Wa2, ba2):
    h0 = nfeats.reshape(_N, _D)
    src = edge_index[0].astype(jnp.int32).reshape(_NW, _NCH, _CHUNK)
    dst = edge_index[1].astype(jnp.int32).reshape(_NW, _NCH, _CHUNK)
    ep = jnp.concatenate(
        [efeats.reshape(_E, _ED),
         jnp.ones((_E, _TDW - _ED), jnp.float32)], axis=1,
    ).reshape(_NW * _NCH, _CHUNK, _TDW)
    z128 = jnp.zeros((_N, _D), jnp.float32)
    z32 = jnp.zeros((_N, _TDW), jnp.float32)

    s1, td = _scatter1(h0, src, dst, ep, z128, z32)
    s1 = s1.reshape(_NC, _N, _D)
    td = td.reshape(_NC, _N, _TDW)
    h1 = _tc_apply(s1, td, h0, Wm1, bm1.reshape(1, _D), Wa1, ba1.reshape(1, _D))
    s2 = _scatter2(h1, src, dst, z128).reshape(_NC, _N, _D)
    h2 = _tc_apply(s2, td, h1, Wm2, bm2.reshape(1, _D), Wa2, ba2.reshape(1, _D))
    return h2


# speedup vs baseline: 3.9808x; 3.9808x over previous
import functools

import jax
import jax.numpy as jnp
from jax import lax
from jax.experimental import pallas as pl
from jax.experimental.pallas import tpu as pltpu
from jax.experimental.pallas import tpu_sc as plsc

_N = 10000
_E = 320000
_D = 128
_HD = 64
_ED = 16

_NC = 2
_NS = 16
_NW = _NC * _NS
_EPT = _E // _NS
_CHUNK = 80
_NCH = _EPT // _CHUNK
_NP = 10240
_RPT = _NP // _NS

_mesh = plsc.VectorSubcoreMesh(core_axis_name="c", subcore_axis_name="s")
_sc_params = pltpu.CompilerParams(use_tc_tiling_on_sc=False)


def _sc_scatter1(h2_hbm, gsrc_hbm, dst_hbm, ep_hbm, ones_hbm, z64_hbm,
                 z16_hbm, s_out, td_out,
                 src_v, dst_v, rows_v, ep_v, sem, acc_s, acc_td):
    c = lax.axis_index("c")
    s = lax.axis_index("s")
    w = c * _NS + s
    r0 = s * _RPT
    pltpu.sync_copy(z64_hbm.at[pl.ds(r0, _RPT)], acc_s.at[pl.ds(r0, _RPT)])
    pltpu.sync_copy(z16_hbm.at[pl.ds(r0, _RPT)], acc_td.at[pl.ds(r0, _RPT)])
    pltpu.sync_copy(gsrc_hbm.at[w], src_v)
    pltpu.sync_copy(dst_hbm.at[s], dst_v)
    @pl.when(c == 1)
    def _():
        pltpu.sync_copy(ones_hbm, ep_v)
    plsc.subcore_barrier()

    def chunk(j, carry):
        pltpu.async_copy(h2_hbm.at[src_v.at[j]], rows_v, sem).wait()
        pltpu.sync_copy(rows_v, acc_s.at[dst_v.at[j]], add=True)

        @pl.when(c == 0)
        def _():
            pltpu.async_copy(ep_hbm.at[s * _NCH + j], ep_v, sem).wait()

        pltpu.sync_copy(ep_v, acc_td.at[dst_v.at[j]], add=True)
        return carry

    lax.fori_loop(0, _NCH, chunk, 0)
    plsc.subcore_barrier()
    pltpu.sync_copy(acc_s.at[pl.ds(r0, _RPT)], s_out.at[w])
    pltpu.sync_copy(acc_td.at[pl.ds(r0, _RPT)], td_out.at[w])


def _sc_scatter2(h2_hbm, gsrc_hbm, dst_hbm, z64_hbm,
                 s_out,
                 src_v, dst_v, rows_v, sem, acc_s):
    c = lax.axis_index("c")
    s = lax.axis_index("s")
    w = c * _NS + s
    r0 = s * _RPT
    pltpu.sync_copy(z64_hbm.at[pl.ds(r0, _RPT)], acc_s.at[pl.ds(r0, _RPT)])
    pltpu.sync_copy(gsrc_hbm.at[w], src_v)
    pltpu.sync_copy(dst_hbm.at[s], dst_v)
    plsc.subcore_barrier()

    def chunk(j, carry):
        pltpu.async_copy(h2_hbm.at[src_v.at[j]], rows_v, sem).wait()
        pltpu.sync_copy(rows_v, acc_s.at[dst_v.at[j]], add=True)
        return carry

    lax.fori_loop(0, _NCH, chunk, 0)
    plsc.subcore_barrier()
    pltpu.sync_copy(acc_s.at[pl.ds(r0, _RPT)], s_out.at[w])


_scatter1 = functools.partial(
    pl.kernel,
    out_type=(jax.ShapeDtypeStruct((_NW, _RPT, _HD), jnp.float32),
              jax.ShapeDtypeStruct((_NW, _RPT, _ED), jnp.float32)),
    mesh=_mesh,
    compiler_params=_sc_params,
    scratch_types=(
        pltpu.VMEM((_NCH, _CHUNK), jnp.int32),
        pltpu.VMEM((_NCH, _CHUNK), jnp.int32),
        pltpu.VMEM((_CHUNK, _HD), jnp.float32),
        pltpu.VMEM((_CHUNK, _ED), jnp.float32),
        pltpu.SemaphoreType.DMA,
        pltpu.VMEM_SHARED((_NP, _HD), jnp.float32),
        pltpu.VMEM_SHARED((_NP, _ED), jnp.float32),
    ),
)(_sc_scatter1)

_scatter2 = functools.partial(
    pl.kernel,
    out_type=jax.ShapeDtypeStruct((_NW, _RPT, _HD), jnp.float32),
    mesh=_mesh,
    compiler_params=_sc_params,
    scratch_types=(
        pltpu.VMEM((_NCH, _CHUNK), jnp.int32),
        pltpu.VMEM((_NCH, _CHUNK), jnp.int32),
        pltpu.VMEM((_CHUNK, _HD), jnp.float32),
        pltpu.SemaphoreType.DMA,
        pltpu.VMEM_SHARED((_NP, _HD), jnp.float32),
    ),
)(_sc_scatter2)


def _tc_apply_body(sp_ref, tdp_ref, h_ref, wm_ref, bm_ref, wa_ref, ba_ref,
                   out_ref):
    s_lo = sp_ref[0]
    s_hi = sp_ref[1]
    t = tdp_ref[0]
    deg = tdp_ref[1][:, 0:1]
    agg = (jnp.dot(s_lo, wm_ref[:_HD, :], preferred_element_type=jnp.float32)
           + jnp.dot(s_hi, wm_ref[_HD:_D, :],
                     preferred_element_type=jnp.float32)
           + jnp.dot(t, wm_ref[_D:, :], preferred_element_type=jnp.float32)
           + deg * bm_ref[...])
    h_neigh = agg / jnp.maximum(deg, 1.0)
    out = (jnp.dot(h_ref[...], wa_ref[:_D, :],
                   preferred_element_type=jnp.float32)
           + jnp.dot(h_neigh, wa_ref[_D:, :],
                     preferred_element_type=jnp.float32)
           + ba_ref[...])
    out_ref[...] = jnp.maximum(out, 0.0)


_TC_B = 2000


def _tc_apply(s_part, td_part, h, wm, bm, wa, ba):
    grid = (_N // _TC_B,)
    return pl.pallas_call(
        _tc_apply_body,
        grid=grid,
        in_specs=[
            pl.BlockSpec((_NC, _TC_B, _HD), lambda i: (0, i, 0)),
            pl.BlockSpec((_NC, _TC_B, _ED), lambda i: (0, i, 0)),
            pl.BlockSpec((_TC_B, _D), lambda i: (i, 0)),
            pl.BlockSpec((_D + _ED, _D), lambda i: (0, 0)),
            pl.BlockSpec((1, _D), lambda i: (0, 0)),
            pl.BlockSpec((2 * _D, _D), lambda i: (0, 0)),
            pl.BlockSpec((1, _D), lambda i: (0, 0)),
        ],
        out_specs=pl.BlockSpec((_TC_B, _D), lambda i: (i, 0)),
        out_shape=jax.ShapeDtypeStruct((_N, _D), jnp.float32),
    )(s_part, td_part, h, wm, bm, wa, ba)


def kernel(nfeats, edge_index, efeats, Wm1, bm1, Wa1, ba1, Wm2, bm2, Wa2, ba2):
    h0 = nfeats.reshape(_N, _D)
    src = edge_index[0].astype(jnp.int32)
    dst = edge_index[1].astype(jnp.int32)
    gsrc = (2 * src[None, :] + jnp.arange(_NC, dtype=jnp.int32)[:, None]
            ).reshape(_NW, _NCH, _CHUNK)
    dst3 = dst.reshape(_NS, _NCH, _CHUNK)
    ep = efeats.reshape(_NS * _NCH, _CHUNK, _ED)
    ones = jnp.ones((_CHUNK, _ED), jnp.float32)
    z64 = jnp.zeros((_NP, _HD), jnp.float32)
    z16 = jnp.zeros((_NP, _ED), jnp.float32)

    s1, td = _scatter1(h0.reshape(2 * _N, _HD), gsrc, dst3, ep, ones,
                       z64, z16)
    s1 = s1.reshape(_NC, _NP, _HD)
    td = td.reshape(_NC, _NP, _ED)
    h1 = _tc_apply(s1, td, h0, Wm1, bm1.reshape(1, _D), Wa1, ba1.reshape(1, _D))
    s2 = _scatter2(h1.reshape(2 * _N, _HD), gsrc, dst3,
                   z64).reshape(_NC, _NP, _HD)
    h2 = _tc_apply(s2, td, h1, Wm2, bm2.reshape(1, _D), Wa2, ba2.reshape(1, _D))
    return h2


# speedup vs baseline: 7.4148x; 1.8627x over previous
import functools

import jax
import jax.numpy as jnp
from jax import lax
from jax.experimental import pallas as pl
from jax.experimental.pallas import tpu as pltpu
from jax.experimental.pallas import tpu_sc as plsc

_N = 10000
_E = 320000
_D = 128
_HD = 64
_ED = 16

_NC = 2
_NS = 16
_NW = _NC * _NS
_EPT = _E // _NS
_CHUNK = 80
_NCH = _EPT // _CHUNK
_NB = 2
_NP = 10240
_RPT = _NP // _NS

_mesh = plsc.VectorSubcoreMesh(core_axis_name="c", subcore_axis_name="s")
_sc_params = pltpu.CompilerParams(use_tc_tiling_on_sc=False)


def _sc_scatter1(h2_hbm, gsrc_hbm, dst_hbm, ep_hbm, ones_hbm, z64_hbm,
                 z16_hbm, s_out, td_out,
                 src_v, dst_v, rows_v, ep_v, gsem0, gsem1, esem0, esem1,
                 acc_s, acc_td):
    c = lax.axis_index("c")
    s = lax.axis_index("s")
    w = c * _NS + s
    r0 = s * _RPT
    gsems = (gsem0, gsem1)
    esems = (esem0, esem1)
    pltpu.sync_copy(gsrc_hbm.at[w], src_v)
    pltpu.sync_copy(dst_hbm.at[s], dst_v)
    for b in range(_NB):
        pltpu.async_copy(h2_hbm.at[src_v.at[b]], rows_v.at[b], gsems[b])

    @pl.when(c == 0)
    def _():
        for b in range(_NB):
            pltpu.async_copy(ep_hbm.at[s * _NCH + b], ep_v.at[b], esems[b])

    @pl.when(c == 1)
    def _():
        for b in range(_NB):
            pltpu.sync_copy(ones_hbm, ep_v.at[b])

    pltpu.sync_copy(z64_hbm.at[pl.ds(r0, _RPT)], acc_s.at[pl.ds(r0, _RPT)])
    pltpu.sync_copy(z16_hbm.at[pl.ds(r0, _RPT)], acc_td.at[pl.ds(r0, _RPT)])
    plsc.subcore_barrier()

    @pl.loop(0, _NCH, step=_NB)
    def _(j0):
        for b in range(_NB):
            j = j0 + b
            jn = j + _NB
            pltpu.make_async_copy(h2_hbm.at[pl.ds(0, _CHUNK)], rows_v.at[b],
                                  gsems[b]).wait()
            pltpu.sync_copy(rows_v.at[b], acc_s.at[dst_v.at[j]], add=True)

            @pl.when(jn < _NCH)
            def _():
                pltpu.async_copy(h2_hbm.at[src_v.at[jn]], rows_v.at[b],
                                 gsems[b])

            @pl.when(c == 0)
            def _():
                pltpu.make_async_copy(ep_hbm.at[0], ep_v.at[b],
                                      esems[b]).wait()

            pltpu.sync_copy(ep_v.at[b], acc_td.at[dst_v.at[j]], add=True)

            @pl.when((c == 0) & (jn < _NCH))
            def _():
                pltpu.async_copy(ep_hbm.at[s * _NCH + jn], ep_v.at[b],
                                 esems[b])

    plsc.subcore_barrier()
    pltpu.sync_copy(acc_s.at[pl.ds(r0, _RPT)], s_out.at[w])
    pltpu.sync_copy(acc_td.at[pl.ds(r0, _RPT)], td_out.at[w])


def _sc_scatter2(h2_hbm, gsrc_hbm, dst_hbm, z64_hbm,
                 s_out,
                 src_v, dst_v, rows_v, gsem0, gsem1, acc_s):
    c = lax.axis_index("c")
    s = lax.axis_index("s")
    w = c * _NS + s
    r0 = s * _RPT
    gsems = (gsem0, gsem1)
    pltpu.sync_copy(gsrc_hbm.at[w], src_v)
    pltpu.sync_copy(dst_hbm.at[s], dst_v)
    for b in range(_NB):
        pltpu.async_copy(h2_hbm.at[src_v.at[b]], rows_v.at[b], gsems[b])
    pltpu.sync_copy(z64_hbm.at[pl.ds(r0, _RPT)], acc_s.at[pl.ds(r0, _RPT)])
    plsc.subcore_barrier()

    @pl.loop(0, _NCH, step=_NB)
    def _(j0):
        for b in range(_NB):
            j = j0 + b
            jn = j + _NB
            pltpu.make_async_copy(h2_hbm.at[pl.ds(0, _CHUNK)], rows_v.at[b],
                                  gsems[b]).wait()
            pltpu.sync_copy(rows_v.at[b], acc_s.at[dst_v.at[j]], add=True)

            @pl.when(jn < _NCH)
            def _():
                pltpu.async_copy(h2_hbm.at[src_v.at[jn]], rows_v.at[b],
                                 gsems[b])

    plsc.subcore_barrier()
    pltpu.sync_copy(acc_s.at[pl.ds(r0, _RPT)], s_out.at[w])


_scatter1 = functools.partial(
    pl.kernel,
    out_type=(jax.ShapeDtypeStruct((_NW, _RPT, _HD), jnp.float32),
              jax.ShapeDtypeStruct((_NW, _RPT, _ED), jnp.float32)),
    mesh=_mesh,
    compiler_params=_sc_params,
    scratch_types=(
        pltpu.VMEM((_NCH, _CHUNK), jnp.int32),
        pltpu.VMEM((_NCH, _CHUNK), jnp.int32),
        pltpu.VMEM((_NB, _CHUNK, _HD), jnp.float32),
        pltpu.VMEM((_NB, _CHUNK, _ED), jnp.float32),
        pltpu.SemaphoreType.DMA,
        pltpu.SemaphoreType.DMA,
        pltpu.SemaphoreType.DMA,
        pltpu.SemaphoreType.DMA,
        pltpu.VMEM_SHARED((_NP, _HD), jnp.float32),
        pltpu.VMEM_SHARED((_NP, _ED), jnp.float32),
    ),
)(_sc_scatter1)

_scatter2 = functools.partial(
    pl.kernel,
    out_type=jax.ShapeDtypeStruct((_NW, _RPT, _HD), jnp.float32),
    mesh=_mesh,
    compiler_params=_sc_params,
    scratch_types=(
        pltpu.VMEM((_NCH, _CHUNK), jnp.int32),
        pltpu.VMEM((_NCH, _CHUNK), jnp.int32),
        pltpu.VMEM((_NB, _CHUNK, _HD), jnp.float32),
        pltpu.SemaphoreType.DMA,
        pltpu.SemaphoreType.DMA,
        pltpu.VMEM_SHARED((_NP, _HD), jnp.float32),
    ),
)(_sc_scatter2)


def _tc_apply_body(sp_ref, tdp_ref, h_ref, wm_ref, bm_ref, wa_ref, ba_ref,
                   out_ref):
    s_lo = sp_ref[0]
    s_hi = sp_ref[1]
    t = tdp_ref[0]
    deg = tdp_ref[1][:, 0:1]
    agg = (jnp.dot(s_lo, wm_ref[:_HD, :], preferred_element_type=jnp.float32)
           + jnp.dot(s_hi, wm_ref[_HD:_D, :],
                     preferred_element_type=jnp.float32)
           + jnp.dot(t, wm_ref[_D:, :], preferred_element_type=jnp.float32)
           + deg * bm_ref[...])
    h_neigh = agg / jnp.maximum(deg, 1.0)
    out = (jnp.dot(h_ref[...], wa_ref[:_D, :],
                   preferred_element_type=jnp.float32)
           + jnp.dot(h_neigh, wa_ref[_D:, :],
                     preferred_element_type=jnp.float32)
           + ba_ref[...])
    out_ref[...] = jnp.maximum(out, 0.0)


_TC_B = 2000


def _tc_apply(s_part, td_part, h, wm, bm, wa, ba):
    grid = (_N // _TC_B,)
    return pl.pallas_call(
        _tc_apply_body,
        grid=grid,
        in_specs=[
            pl.BlockSpec((_NC, _TC_B, _HD), lambda i: (0, i, 0)),
            pl.BlockSpec((_NC, _TC_B, _ED), lambda i: (0, i, 0)),
            pl.BlockSpec((_TC_B, _D), lambda i: (i, 0)),
            pl.BlockSpec((_D + _ED, _D), lambda i: (0, 0)),
            pl.BlockSpec((1, _D), lambda i: (0, 0)),
            pl.BlockSpec((2 * _D, _D), lambda i: (0, 0)),
            pl.BlockSpec((1, _D), lambda i: (0, 0)),
        ],
        out_specs=pl.BlockSpec((_TC_B, _D), lambda i: (i, 0)),
        out_shape=jax.ShapeDtypeStruct((_N, _D), jnp.float32),
    )(s_part, td_part, h, wm, bm, wa, ba)


def kernel(nfeats, edge_index, efeats, Wm1, bm1, Wa1, ba1, Wm2, bm2, Wa2, ba2):
    h0 = nfeats.reshape(_N, _D)
    src = edge_index[0].astype(jnp.int32)
    dst = edge_index[1].astype(jnp.int32)
    gsrc = (2 * src[None, :] + jnp.arange(_NC, dtype=jnp.int32)[:, None]
            ).reshape(_NW, _NCH, _CHUNK)
    dst3 = dst.reshape(_NS, _NCH, _CHUNK)
    ep = efeats.reshape(_NS * _NCH, _CHUNK, _ED)
    ones = jnp.ones((_CHUNK, _ED), jnp.float32)
    z64 = jnp.zeros((_NP, _HD), jnp.float32)
    z16 = jnp.zeros((_NP, _ED), jnp.float32)

    s1, td = _scatter1(h0.reshape(2 * _N, _HD), gsrc, dst3, ep, ones,
                       z64, z16)
    s1 = s1.reshape(_NC, _NP, _HD)
    td = td.reshape(_NC, _NP, _ED)
    h1 = _tc_apply(s1, td, h0, Wm1, bm1.reshape(1, _D), Wa1, ba1.reshape(1, _D))
    s2 = _scatter2(h1.reshape(2 * _N, _HD), gsrc, dst3,
                   z64).reshape(_NC, _NP, _HD)
    h2 = _tc_apply(s2, td, h1, Wm2, bm2.reshape(1, _D), Wa2, ba2.reshape(1, _D))
    return h2


# speedup vs baseline: 8.5932x; 1.1589x over previous
import functools

import jax
import jax.numpy as jnp
from jax import lax
from jax.experimental import pallas as pl
from jax.experimental.pallas import tpu as pltpu
from jax.experimental.pallas import tpu_sc as plsc

_N = 10000
_E = 320000
_D = 128
_HD = 64
_ED = 16

_NC = 2
_NS = 16
_NW = _NC * _NS
_EPT = _E // _NS
_CHUNK = 80
_NCH = _EPT // _CHUNK
_NB = 5
_NP = 10240
_RPT = _NP // _NS

_mesh = plsc.VectorSubcoreMesh(core_axis_name="c", subcore_axis_name="s")
_sc_params = pltpu.CompilerParams(use_tc_tiling_on_sc=False)


def _prep_body(ei_ref, et_ref, gsrc_ref, dst_ref, ep_ref):
    ei = ei_ref[...]
    src2 = 2 * jnp.broadcast_to(ei[0:1, :], ei.shape)
    gsrc_ref[...] = src2 + lax.broadcasted_iota(jnp.int32, ei.shape, 0)
    dst_ref[...] = ei[1:2, :]
    ep_ref[...] = et_ref[...].T


_PREP_BE = 12800


def _prep(edge_index, et):
    grid = (_E // _PREP_BE,)
    return pl.pallas_call(
        _prep_body,
        grid=grid,
        in_specs=[
            pl.BlockSpec((2, _PREP_BE), lambda i: (0, i)),
            pl.BlockSpec((_ED, _PREP_BE), lambda i: (0, i)),
        ],
        out_specs=[
            pl.BlockSpec((2, _PREP_BE), lambda i: (0, i)),
            pl.BlockSpec((1, _PREP_BE), lambda i: (0, i)),
            pl.BlockSpec((_PREP_BE, _ED), lambda i: (i, 0)),
        ],
        out_shape=[
            jax.ShapeDtypeStruct((2, _E), jnp.int32),
            jax.ShapeDtypeStruct((1, _E), jnp.int32),
            jax.ShapeDtypeStruct((_E, _ED), jnp.float32),
        ],
    )(edge_index, et)


def _sc_scatter1(h2_hbm, gsrc_hbm, dst_hbm, ep_hbm, ones_hbm, z64_hbm,
                 z16_hbm, s_out, td_out,
                 src_v, dst_v, rows_v, ep_v, gsems, esems, acc_s, acc_td):
    c = lax.axis_index("c")
    s = lax.axis_index("s")
    w = c * _NS + s
    r0 = s * _RPT
    pltpu.sync_copy(gsrc_hbm.at[w], src_v)
    pltpu.sync_copy(dst_hbm.at[s], dst_v)
    for b in range(_NB):
        pltpu.async_copy(h2_hbm.at[src_v.at[b]], rows_v.at[b], gsems[b])

    @pl.when(c == 0)
    def _():
        for b in range(_NB):
            pltpu.async_copy(ep_hbm.at[s * _NCH + b], ep_v.at[b], esems[b])

    @pl.when(c == 1)
    def _():
        for b in range(_NB):
            pltpu.sync_copy(ones_hbm, ep_v.at[b])

    pltpu.sync_copy(z64_hbm.at[pl.ds(r0, _RPT)], acc_s.at[pl.ds(r0, _RPT)])
    pltpu.sync_copy(z16_hbm.at[pl.ds(r0, _RPT)], acc_td.at[pl.ds(r0, _RPT)])
    plsc.subcore_barrier()

    @pl.loop(0, _NCH, step=_NB)
    def _(j0):
        for b in range(_NB):
            j = j0 + b
            jn = j + _NB
            pltpu.make_async_copy(h2_hbm.at[pl.ds(0, _CHUNK)], rows_v.at[b],
                                  gsems[b]).wait()
            pltpu.sync_copy(rows_v.at[b], acc_s.at[dst_v.at[j]], add=True)

            @pl.when(jn < _NCH)
            def _():
                pltpu.async_copy(h2_hbm.at[src_v.at[jn]], rows_v.at[b],
                                 gsems[b])

            @pl.when(c == 0)
            def _():
                pltpu.make_async_copy(ep_hbm.at[0], ep_v.at[b],
                                      esems[b]).wait()

            pltpu.sync_copy(ep_v.at[b], acc_td.at[dst_v.at[j]], add=True)

            @pl.when((c == 0) & (jn < _NCH))
            def _():
                pltpu.async_copy(ep_hbm.at[s * _NCH + jn], ep_v.at[b],
                                 esems[b])

    plsc.subcore_barrier()
    pltpu.sync_copy(acc_s.at[pl.ds(r0, _RPT)],
                    s_out.at[pl.ds(r0, _RPT), pl.ds(c * _HD, _HD)])
    pltpu.sync_copy(acc_td.at[pl.ds(r0, _RPT)], td_out.at[w])


def _sc_scatter2(h2_hbm, gsrc_hbm, dst_hbm, z64_hbm,
                 s_out,
                 src_v, dst_v, rows_v, gsems, acc_s):
    c = lax.axis_index("c")
    s = lax.axis_index("s")
    w = c * _NS + s
    r0 = s * _RPT
    pltpu.sync_copy(gsrc_hbm.at[w], src_v)
    pltpu.sync_copy(dst_hbm.at[s], dst_v)
    for b in range(_NB):
        pltpu.async_copy(h2_hbm.at[src_v.at[b]], rows_v.at[b], gsems[b])
    pltpu.sync_copy(z64_hbm.at[pl.ds(r0, _RPT)], acc_s.at[pl.ds(r0, _RPT)])
    plsc.subcore_barrier()

    @pl.loop(0, _NCH, step=_NB)
    def _(j0):
        for b in range(_NB):
            j = j0 + b
            jn = j + _NB
            pltpu.make_async_copy(h2_hbm.at[pl.ds(0, _CHUNK)], rows_v.at[b],
                                  gsems[b]).wait()
            pltpu.sync_copy(rows_v.at[b], acc_s.at[dst_v.at[j]], add=True)

            @pl.when(jn < _NCH)
            def _():
                pltpu.async_copy(h2_hbm.at[src_v.at[jn]], rows_v.at[b],
                                 gsems[b])

    plsc.subcore_barrier()
    pltpu.sync_copy(acc_s.at[pl.ds(r0, _RPT)],
                    s_out.at[pl.ds(r0, _RPT), pl.ds(c * _HD, _HD)])


_scatter1 = functools.partial(
    pl.kernel,
    out_type=(jax.ShapeDtypeStruct((_NP, _D), jnp.float32),
              jax.ShapeDtypeStruct((_NW, _RPT, _ED), jnp.float32)),
    mesh=_mesh,
    compiler_params=_sc_params,
    scratch_types=(
        pltpu.VMEM((_NCH, _CHUNK), jnp.int32),
        pltpu.VMEM((_NCH, _CHUNK), jnp.int32),
        pltpu.VMEM((_NB, _CHUNK, _HD), jnp.float32),
        pltpu.VMEM((_NB, _CHUNK, _ED), jnp.float32),
        tuple(pltpu.SemaphoreType.DMA for _ in range(_NB)),
        tuple(pltpu.SemaphoreType.DMA for _ in range(_NB)),
        pltpu.VMEM_SHARED((_NP, _HD), jnp.float32),
        pltpu.VMEM_SHARED((_NP, _ED), jnp.float32),
    ),
)(_sc_scatter1)

_scatter2 = functools.partial(
    pl.kernel,
    out_type=jax.ShapeDtypeStruct((_NP, _D), jnp.float32),
    mesh=_mesh,
    compiler_params=_sc_params,
    scratch_types=(
        pltpu.VMEM((_NCH, _CHUNK), jnp.int32),
        pltpu.VMEM((_NCH, _CHUNK), jnp.int32),
        pltpu.VMEM((_NB, _CHUNK, _HD), jnp.float32),
        tuple(pltpu.SemaphoreType.DMA for _ in range(_NB)),
        pltpu.VMEM_SHARED((_NP, _HD), jnp.float32),
    ),
)(_sc_scatter2)


def _tc_apply_body(s_ref, tdp_ref, h_ref, wm_ref, bm_ref, wa_ref, ba_ref,
                   out_ref):
    t = tdp_ref[0]
    deg = tdp_ref[1][:, 0:1]
    agg = (jnp.dot(s_ref[...], wm_ref[:_D, :],
                   preferred_element_type=jnp.float32)
           + jnp.dot(t, wm_ref[_D:, :], preferred_element_type=jnp.float32)
           + deg * bm_ref[...])
    h_neigh = agg / jnp.maximum(deg, 1.0)
    out = (jnp.dot(h_ref[...], wa_ref[:_D, :],
                   preferred_element_type=jnp.float32)
           + jnp.dot(h_neigh, wa_ref[_D:, :],
                     preferred_element_type=jnp.float32)
           + ba_ref[...])
    out_ref[...] = jnp.maximum(out, 0.0)


_TC_B = 2000


def _tc_apply(s, td_part, h, wm, bm, wa, ba):
    grid = (_N // _TC_B,)
    return pl.pallas_call(
        _tc_apply_body,
        grid=grid,
        in_specs=[
            pl.BlockSpec((_TC_B, _D), lambda i: (i, 0)),
            pl.BlockSpec((_NC, _TC_B, _ED), lambda i: (0, i, 0)),
            pl.BlockSpec((_TC_B, _D), lambda i: (i, 0)),
            pl.BlockSpec((_D + _ED, _D), lambda i: (0, 0)),
            pl.BlockSpec((1, _D), lambda i: (0, 0)),
            pl.BlockSpec((2 * _D, _D), lambda i: (0, 0)),
            pl.BlockSpec((1, _D), lambda i: (0, 0)),
        ],
        out_specs=pl.BlockSpec((_TC_B, _D), lambda i: (i, 0)),
        out_shape=jax.ShapeDtypeStruct((_N, _D), jnp.float32),
    )(s, td_part, h, wm, bm, wa, ba)


def kernel(nfeats, edge_index, efeats, Wm1, bm1, Wa1, ba1, Wm2, bm2, Wa2, ba2):
    h0 = nfeats.reshape(_N, _D)
    ei = edge_index.astype(jnp.int32)
    et = efeats.reshape(_E, _ED).T
    gsrc2, dst2, ep2 = _prep(ei, et)
    gsrc = gsrc2.reshape(_NW, _NCH, _CHUNK)
    dst3 = dst2.reshape(_NS, _NCH, _CHUNK)
    ep = ep2.reshape(_NS * _NCH, _CHUNK, _ED)
    ones = jnp.ones((_CHUNK, _ED), jnp.float32)
    z64 = jnp.zeros((_NP, _HD), jnp.float32)
    z16 = jnp.zeros((_NP, _ED), jnp.float32)

    s1, td = _scatter1(h0.reshape(2 * _N, _HD), gsrc, dst3, ep, ones,
                       z64, z16)
    td = td.reshape(_NC, _NP, _ED)
    h1 = _tc_apply(s1, td, h0, Wm1, bm1.reshape(1, _D), Wa1,
                   ba1.reshape(1, _D))
    s2 = _scatter2(h1.reshape(2 * _N, _HD), gsrc, dst3, z64)
    h2 = _tc_apply(s2, td, h1, Wm2, bm2.reshape(1, _D), Wa2,
                   ba2.reshape(1, _D))
    return h2


# speedup vs baseline: 9.4377x; 1.0983x over previous
import functools

import jax
import jax.numpy as jnp
from jax import lax
from jax.experimental import pallas as pl
from jax.experimental.pallas import tpu as pltpu
from jax.experimental.pallas import tpu_sc as plsc

_N = 10000
_E = 320000
_D = 128
_HD = 64
_ED = 16

_NC = 2
_NS = 16
_NW = _NC * _NS
_EPT = _E // _NS
_CHUNK = 80
_NCH = _EPT // _CHUNK
_NB = 5
_NP = 10240
_RPT = _NP // _NS

_mesh = plsc.VectorSubcoreMesh(core_axis_name="c", subcore_axis_name="s")
_sc_params = pltpu.CompilerParams(use_tc_tiling_on_sc=False)


def _sc_scatter_td(dst_hbm, ep_hbm, ones_hbm, z16_hbm,
                   td_out,
                   dst_v, ep_v, esems, acc_td):
    c = lax.axis_index("c")
    s = lax.axis_index("s")
    w = c * _NS + s
    r0 = s * _RPT
    pltpu.sync_copy(dst_hbm.at[s], dst_v)

    @pl.when(c == 0)
    def _():
        for b in range(_NB):
            pltpu.async_copy(ep_hbm.at[s * _NCH + b], ep_v.at[b], esems[b])

    @pl.when(c == 1)
    def _():
        for b in range(_NB):
            pltpu.sync_copy(ones_hbm, ep_v.at[b])

    pltpu.sync_copy(z16_hbm.at[pl.ds(r0, _RPT)], acc_td.at[pl.ds(r0, _RPT)])
    plsc.subcore_barrier()

    @pl.loop(0, _NCH, step=_NB)
    def _(j0):
        for b in range(_NB):
            j = j0 + b
            jn = j + _NB

            @pl.when(c == 0)
            def _():
                pltpu.make_async_copy(ep_hbm.at[0], ep_v.at[b],
                                      esems[b]).wait()

            pltpu.sync_copy(ep_v.at[b], acc_td.at[dst_v.at[j]], add=True)

            @pl.when((c == 0) & (jn < _NCH))
            def _():
                pltpu.async_copy(ep_hbm.at[s * _NCH + jn], ep_v.at[b],
                                 esems[b])

    plsc.subcore_barrier()
    pltpu.sync_copy(acc_td.at[pl.ds(r0, _RPT)], td_out.at[w])


def _sc_scatter2(h2_hbm, gsrc_hbm, dst_hbm, z64_hbm,
                 s_out,
                 src_v, dst_v, rows_v, gsems, acc_s):
    c = lax.axis_index("c")
    s = lax.axis_index("s")
    w = c * _NS + s
    r0 = s * _RPT
    pltpu.sync_copy(gsrc_hbm.at[w], src_v)
    pltpu.sync_copy(dst_hbm.at[s], dst_v)
    for b in range(_NB):
        pltpu.async_copy(h2_hbm.at[src_v.at[b]], rows_v.at[b], gsems[b])
    pltpu.sync_copy(z64_hbm.at[pl.ds(r0, _RPT)], acc_s.at[pl.ds(r0, _RPT)])
    plsc.subcore_barrier()

    @pl.loop(0, _NCH, step=_NB)
    def _(j0):
        for b in range(_NB):
            j = j0 + b
            jn = j + _NB
            pltpu.make_async_copy(h2_hbm.at[pl.ds(0, _CHUNK)], rows_v.at[b],
                                  gsems[b]).wait()
            pltpu.sync_copy(rows_v.at[b], acc_s.at[dst_v.at[j]], add=True)

            @pl.when(jn < _NCH)
            def _():
                pltpu.async_copy(h2_hbm.at[src_v.at[jn]], rows_v.at[b],
                                 gsems[b])

    plsc.subcore_barrier()
    pltpu.sync_copy(acc_s.at[pl.ds(r0, _RPT)],
                    s_out.at[pl.ds(r0, _RPT), pl.ds(c * _HD, _HD)])


_scatter_td = functools.partial(
    pl.kernel,
    out_type=jax.ShapeDtypeStruct((_NW, _RPT, _ED), jnp.float32),
    mesh=_mesh,
    compiler_params=_sc_params,
    scratch_types=(
        pltpu.VMEM((_NCH, _CHUNK), jnp.int32),
        pltpu.VMEM((_NB, _CHUNK, _ED), jnp.float32),
        tuple(pltpu.SemaphoreType.DMA for _ in range(_NB)),
        pltpu.VMEM_SHARED((_NP, _ED), jnp.float32),
    ),
)(_sc_scatter_td)

_scatter2 = functools.partial(
    pl.kernel,
    out_type=jax.ShapeDtypeStruct((_NP, _D), jnp.float32),
    mesh=_mesh,
    compiler_params=_sc_params,
    scratch_types=(
        pltpu.VMEM((_NCH, _CHUNK), jnp.int32),
        pltpu.VMEM((_NCH, _CHUNK), jnp.int32),
        pltpu.VMEM((_NB, _CHUNK, _HD), jnp.float32),
        tuple(pltpu.SemaphoreType.DMA for _ in range(_NB)),
        pltpu.VMEM_SHARED((_NP, _HD), jnp.float32),
    ),
)(_sc_scatter2)


def _tc_apply_body(s_ref, tdp_ref, h_ref, wm_ref, bm_ref, wa_ref, ba_ref,
                   out_ref):
    t = tdp_ref[0]
    deg = tdp_ref[1][:, 0:1]
    agg = (jnp.dot(s_ref[...], wm_ref[:_D, :],
                   preferred_element_type=jnp.float32)
           + jnp.dot(t, wm_ref[_D:, :], preferred_element_type=jnp.float32)
           + deg * bm_ref[...])
    h_neigh = agg / jnp.maximum(deg, 1.0)
    out = (jnp.dot(h_ref[...], wa_ref[:_D, :],
                   preferred_element_type=jnp.float32)
           + jnp.dot(h_neigh, wa_ref[_D:, :],
                     preferred_element_type=jnp.float32)
           + ba_ref[...])
    out_ref[...] = jnp.maximum(out, 0.0)


_TC_B = 2000


def _tc_apply(s, td_part, h, wm, bm, wa, ba):
    grid = (_N // _TC_B,)
    return pl.pallas_call(
        _tc_apply_body,
        grid=grid,
        in_specs=[
            pl.BlockSpec((_TC_B, _D), lambda i: (i, 0)),
            pl.BlockSpec((_NC, _TC_B, _ED), lambda i: (0, i, 0)),
            pl.BlockSpec((_TC_B, _D), lambda i: (i, 0)),
            pl.BlockSpec((_D + _ED, _D), lambda i: (0, 0)),
            pl.BlockSpec((1, _D), lambda i: (0, 0)),
            pl.BlockSpec((2 * _D, _D), lambda i: (0, 0)),
            pl.BlockSpec((1, _D), lambda i: (0, 0)),
        ],
        out_specs=pl.BlockSpec((_TC_B, _D), lambda i: (i, 0)),
        out_shape=jax.ShapeDtypeStruct((_N, _D), jnp.float32),
    )(s, td_part, h, wm, bm, wa, ba)


def kernel(nfeats, edge_index, efeats, Wm1, bm1, Wa1, ba1, Wm2, bm2, Wa2, ba2):
    h0 = nfeats.reshape(_N, _D)
    src = edge_index[0].astype(jnp.int32)
    dst = edge_index[1].astype(jnp.int32)
    gsrc = (2 * src[None, :] + jnp.arange(_NC, dtype=jnp.int32)[:, None]
            ).reshape(_NW, _NCH, _CHUNK)
    dst3 = dst.reshape(_NS, _NCH, _CHUNK)
    ep = efeats.reshape(_NS * _NCH, _CHUNK, _ED)
    ones = jnp.ones((_CHUNK, _ED), jnp.float32)
    z64 = jnp.zeros((_NP, _HD), jnp.float32)
    z16 = jnp.zeros((_NP, _ED), jnp.float32)

    h0r = h0.reshape(2 * _N, _HD)
    s1 = _scatter2(h0r, gsrc, dst3, z64)
    td = _scatter_td(dst3, ep, ones, z16).reshape(_NC, _NP, _ED)
    h1 = _tc_apply(s1, td, h0, Wm1, bm1.reshape(1, _D), Wa1,
                   ba1.reshape(1, _D))
    s2 = _scatter2(h1.reshape(2 * _N, _HD), gsrc, dst3, z64)
    h2 = _tc_apply(s2, td, h1, Wm2, bm2.reshape(1, _D), Wa2,
                   ba2.reshape(1, _D))
    return h2


# speedup vs baseline: 10.9650x; 1.1618x over previous
import functools

import jax
import jax.numpy as jnp
from jax import lax
from jax.experimental import pallas as pl
from jax.experimental.pallas import tpu as pltpu
from jax.experimental.pallas import tpu_sc as plsc

_N = 10000
_E = 320000
_D = 128
_HD = 64
_ED = 16

_NC = 2
_NS = 16
_NW = _NC * _NS
_EPT = _E // _NS
_CHUNK = 80
_NCH = _EPT // _CHUNK
_NB = 5
_NP = 10240
_RPT = _NP // _NS

_mesh = plsc.VectorSubcoreMesh(core_axis_name="c", subcore_axis_name="s")
_sc_params = pltpu.CompilerParams(use_tc_tiling_on_sc=False)


def _sc_scatter_td(dst_hbm, ep_hbm, ones_hbm, z16_hbm,
                   td_out,
                   dst_v, ep_v, esems, acc_td):
    c = lax.axis_index("c")
    s = lax.axis_index("s")
    w = c * _NS + s
    r0 = s * _RPT
    pltpu.sync_copy(dst_hbm.at[s], dst_v)

    @pl.when(c == 0)
    def _():
        for b in range(_NB):
            pltpu.async_copy(ep_hbm.at[s * _NCH + b], ep_v.at[b], esems[b])

    @pl.when(c == 1)
    def _():
        for b in range(_NB):
            pltpu.sync_copy(ones_hbm, ep_v.at[b])

    pltpu.sync_copy(z16_hbm.at[pl.ds(r0, _RPT)], acc_td.at[pl.ds(r0, _RPT)])
    plsc.subcore_barrier()

    @pl.loop(0, _NCH, step=_NB)
    def _(j0):
        for b in range(_NB):
            j = j0 + b
            jn = j + _NB

            @pl.when(c == 0)
            def _():
                pltpu.make_async_copy(ep_hbm.at[0], ep_v.at[b],
                                      esems[b]).wait()

            pltpu.sync_copy(ep_v.at[b], acc_td.at[dst_v.at[j]], add=True)

            @pl.when((c == 0) & (jn < _NCH))
            def _():
                pltpu.async_copy(ep_hbm.at[s * _NCH + jn], ep_v.at[b],
                                 esems[b])

    plsc.subcore_barrier()
    pltpu.sync_copy(acc_td.at[pl.ds(r0, _RPT)], td_out.at[w])


def _sc_scatter2(h2_hbm, gsrc_hbm, dst_hbm, z64_hbm,
                 s_out,
                 src_v, dst_v, rows_v, gsems, acc_s):
    c = lax.axis_index("c")
    s = lax.axis_index("s")
    w = c * _NS + s
    r0 = s * _RPT
    pltpu.sync_copy(gsrc_hbm.at[w], src_v)
    pltpu.sync_copy(dst_hbm.at[s], dst_v)
    for b in range(_NB):
        pltpu.async_copy(h2_hbm.at[src_v.at[b]], rows_v.at[b], gsems[b])
    pltpu.sync_copy(z64_hbm.at[pl.ds(r0, _RPT)], acc_s.at[pl.ds(r0, _RPT)])
    plsc.subcore_barrier()

    @pl.loop(0, _NCH, step=_NB)
    def _(j0):
        for b in range(_NB):
            j = j0 + b
            jn = j + _NB
            pltpu.make_async_copy(h2_hbm.at[pl.ds(0, _CHUNK)], rows_v.at[b],
                                  gsems[b]).wait()
            pltpu.sync_copy(rows_v.at[b], acc_s.at[dst_v.at[j]], add=True)

            @pl.when(jn < _NCH)
            def _():
                pltpu.async_copy(h2_hbm.at[src_v.at[jn]], rows_v.at[b],
                                 gsems[b])

    plsc.subcore_barrier()
    pltpu.sync_copy(acc_s.at[pl.ds(r0, _RPT)],
                    s_out.at[pl.ds(r0, _RPT), pl.ds(c * _HD, _HD)])


_scatter_td = functools.partial(
    pl.kernel,
    out_type=jax.ShapeDtypeStruct((_NW, _RPT, _ED), jnp.float32),
    mesh=_mesh,
    compiler_params=_sc_params,
    scratch_types=(
        pltpu.VMEM((_NCH, _CHUNK), jnp.int32),
        pltpu.VMEM((_NB, _CHUNK, _ED), jnp.float32),
        tuple(pltpu.SemaphoreType.DMA for _ in range(_NB)),
        pltpu.VMEM_SHARED((_NP, _ED), jnp.float32),
    ),
)(_sc_scatter_td)

_scatter2 = functools.partial(
    pl.kernel,
    out_type=jax.ShapeDtypeStruct((_NP, _D), jnp.float32),
    mesh=_mesh,
    compiler_params=_sc_params,
    scratch_types=(
        pltpu.VMEM((_NCH, _CHUNK), jnp.int32),
        pltpu.VMEM((_NCH, _CHUNK), jnp.int32),
        pltpu.VMEM((_NB, _CHUNK, _HD), jnp.float32),
        tuple(pltpu.SemaphoreType.DMA for _ in range(_NB)),
        pltpu.VMEM_SHARED((_NP, _HD), jnp.float32),
    ),
)(_sc_scatter2)


def _tc_apply_body(s_ref, tdp_ref, h_ref, wm_ref, bm_ref, wa_ref, ba_ref,
                   out_ref):
    t = tdp_ref[0]
    deg = tdp_ref[1][:, 0:1]
    agg = (jnp.dot(s_ref[...], wm_ref[:_D, :],
                   preferred_element_type=jnp.float32)
           + jnp.dot(t, wm_ref[_D:, :], preferred_element_type=jnp.float32)
           + deg * bm_ref[...])
    h_neigh = agg / jnp.maximum(deg, 1.0)
    out = (jnp.dot(h_ref[...], wa_ref[:_D, :],
                   preferred_element_type=jnp.float32)
           + jnp.dot(h_neigh, wa_ref[_D:, :],
                     preferred_element_type=jnp.float32)
           + ba_ref[...])
    out_ref[...] = jnp.maximum(out, 0.0)


_TC_B = 2000


def _tc_apply(s, td_part, h, wm, bm, wa, ba):
    grid = (_N // _TC_B,)
    return pl.pallas_call(
        _tc_apply_body,
        grid=grid,
        in_specs=[
            pl.BlockSpec((_TC_B, _D), lambda i: (i, 0)),
            pl.BlockSpec((_NC, _TC_B, _ED), lambda i: (0, i, 0)),
            pl.BlockSpec((_TC_B, _D), lambda i: (i, 0)),
            pl.BlockSpec((_D + _ED, _D), lambda i: (0, 0)),
            pl.BlockSpec((1, _D), lambda i: (0, 0)),
            pl.BlockSpec((2 * _D, _D), lambda i: (0, 0)),
            pl.BlockSpec((1, _D), lambda i: (0, 0)),
        ],
        out_specs=pl.BlockSpec((_TC_B, _D), lambda i: (i, 0)),
        out_shape=jax.ShapeDtypeStruct((_N, _D), jnp.float32),
    )(s, td_part, h, wm, bm, wa, ba)


def kernel(nfeats, edge_index, efeats, Wm1, bm1, Wa1, ba1, Wm2, bm2, Wa2, ba2):
    h0 = nfeats.reshape(_N, _D)
    src = edge_index[0].astype(jnp.int32)
    dst = edge_index[1].astype(jnp.int32)
    gsrc = (2 * src[None, :] + jnp.arange(_NC, dtype=jnp.int32)[:, None]
            ).reshape(_NW, _NCH, _CHUNK)
    dst3 = dst.reshape(_NS, _NCH, _CHUNK)
    ep = efeats.reshape(_NS * _NCH, _CHUNK, _ED)
    ones = jnp.ones((_CHUNK, _ED), jnp.float32)
    z64 = jnp.zeros((_NP, _HD), jnp.float32)
    z16 = jnp.zeros((_NP, _ED), jnp.float32)

    h0r = h0.reshape(2 * _N, _HD)
    s1 = _scatter2(h0r, gsrc, dst3, z64)
    z16d = z16 + s1[0:1, 0:1] * 0.0
    td = _scatter_td(dst3, ep, ones, z16d).reshape(_NC, _NP, _ED)
    h1 = _tc_apply(s1, td, h0, Wm1, bm1.reshape(1, _D), Wa1,
                   ba1.reshape(1, _D))
    s2 = _scatter2(h1.reshape(2 * _N, _HD), gsrc, dst3, z64)
    h2 = _tc_apply(s2, td, h1, Wm2, bm2.reshape(1, _D), Wa2,
                   ba2.reshape(1, _D))
    return h2
